# Initial kernel scaffold; baseline (speedup 1.0000x reference)
import jax
import jax.numpy as jnp
from jax.experimental import pallas as pl


def kernel(x, ei, batch, W1, b1, W2, b2, gamma2, beta2):
    raise NotImplementedError("write your pallas kernel here")


# speedup vs baseline: 12.1505x; 12.1505x over previous
import functools

import jax
import jax.numpy as jnp
from jax import lax
from jax.experimental import pallas as pl
from jax.experimental.pallas import tpu as pltpu
from jax.experimental.pallas import tpu_sc as plsc

N = 10000
E = 320000
D = 128
EPS = 1e-5

NC = 2
NS = 16
NW = NC * NS
EPW = E // NW
CH = 80
NCHUNK = EPW // CH
RPT = 624
RTAIL = N - NS * RPT
NPAD = 10240
DPT = NPAD // NS


def _mesh():
    return plsc.VectorSubcoreMesh(
        core_axis_name="c", subcore_axis_name="s", num_cores=NC, num_subcores=NS
    )


def _sc_degree_body(dst_hbm, degp_hbm, idx_v, ones_v, zero_v, deg_sh):
    cid = lax.axis_index("c")
    sid = lax.axis_index("s")
    wid = cid * NS + sid
    for j in range(CH // 16):
        ones_v[pl.ds(16 * j, 16)] = jnp.ones((16,), jnp.float32)
    for j in range(DPT // 16):
        zero_v[pl.ds(16 * j, 16)] = jnp.zeros((16,), jnp.float32)
    pltpu.sync_copy(zero_v, deg_sh.at[pl.ds(sid * DPT, DPT)])
    plsc.subcore_barrier()

    def step(i, carry):
        base = pl.multiple_of(wid * EPW + i * CH, 8)
        pltpu.sync_copy(dst_hbm.at[pl.ds(base, CH)], idx_v)
        pltpu.sync_copy(ones_v, deg_sh.at[idx_v], add=True)
        return carry

    lax.fori_loop(0, NCHUNK, step, 0)
    plsc.subcore_barrier()
    pltpu.sync_copy(
        deg_sh.at[pl.ds(sid * DPT, DPT)],
        degp_hbm.at[pl.ds(cid * NPAD + sid * DPT, DPT)],
    )


def _sc_segsum_body(y_hbm, src_hbm, dst_hbm, part_hbm, idx_s, idx_d, rows_v, acc_sh):
    cid = lax.axis_index("c")
    sid = lax.axis_index("s")
    wid = cid * NS + sid
    pltpu.sync_copy(y_hbm.at[pl.ds(sid * RPT, RPT)], acc_sh.at[pl.ds(sid * RPT, RPT)])

    @pl.when(sid == 0)
    def _():
        pltpu.sync_copy(
            y_hbm.at[pl.ds(NS * RPT, RTAIL)], acc_sh.at[pl.ds(NS * RPT, RTAIL)]
        )

    plsc.subcore_barrier()

    def step(i, carry):
        base = pl.multiple_of(wid * EPW + i * CH, 8)
        pltpu.sync_copy(src_hbm.at[pl.ds(base, CH)], idx_s)
        pltpu.sync_copy(dst_hbm.at[pl.ds(base, CH)], idx_d)
        pltpu.sync_copy(y_hbm.at[idx_s], rows_v)
        pltpu.sync_copy(rows_v, acc_sh.at[idx_d], add=True)
        return carry

    lax.fori_loop(0, NCHUNK, step, 0)
    plsc.subcore_barrier()
    pltpu.sync_copy(
        acc_sh.at[pl.ds(sid * RPT, RPT)], part_hbm.at[cid, pl.ds(sid * RPT, RPT)]
    )

    @pl.when(sid == 0)
    def _():
        pltpu.sync_copy(
            acc_sh.at[pl.ds(NS * RPT, RTAIL)], part_hbm.at[cid, pl.ds(NS * RPT, RTAIL)]
        )


@functools.cache
def _sc_degree():
    return pl.kernel(
        _sc_degree_body,
        out_type=jax.ShapeDtypeStruct((NC * NPAD,), jnp.float32),
        mesh=_mesh(),
        scratch_types=[
            pltpu.VMEM((CH,), jnp.int32),
            pltpu.VMEM((CH,), jnp.float32),
            pltpu.VMEM((DPT,), jnp.float32),
            pltpu.VMEM_SHARED((NPAD,), jnp.float32),
        ],
    )


@functools.cache
def _sc_segsum():
    return pl.kernel(
        _sc_segsum_body,
        out_type=jax.ShapeDtypeStruct((NC, N, D), jnp.float32),
        mesh=_mesh(),
        scratch_types=[
            pltpu.VMEM((CH,), jnp.int32),
            pltpu.VMEM((CH,), jnp.int32),
            pltpu.VMEM((CH, D), jnp.float32),
            pltpu.VMEM_SHARED((N, D), jnp.float32),
        ],
    )


def _tc_prescale_body(x_ref, w_ref, degp_ref, y_ref, dinv_ref):
    deg = degp_ref[0, :N] + degp_ref[1, :N] + 1.0
    dinv = lax.rsqrt(deg)
    dinv_ref[...] = dinv
    xw = jnp.dot(x_ref[...], w_ref[...], preferred_element_type=jnp.float32)
    y_ref[...] = dinv * xw


def _tc_mid_body(p_ref, y1_ref, dinv_ref, b1_ref, g_ref, be_ref, w2_ref, y2_ref):
    dinv = dinv_ref[...]
    z = dinv * (p_ref[0] + p_ref[1] - y1_ref[...]) + b1_ref[...]
    mean = jnp.mean(z, axis=0, keepdims=True)
    zc = z - mean
    var = jnp.mean(zc * zc, axis=0, keepdims=True)
    h = jnp.maximum(g_ref[...] * zc * lax.rsqrt(var + EPS) + be_ref[...], 0.0)
    hw = jnp.dot(h, w2_ref[...], preferred_element_type=jnp.float32)
    y2_ref[...] = dinv * hw


def _tc_final_body(p_ref, y2_ref, x_ref, dinv_ref, b2_ref, g_ref, be_ref, out_ref):
    z = dinv_ref[...] * (p_ref[0] + p_ref[1] - y2_ref[...]) + b2_ref[...]
    mean = jnp.mean(z, axis=0, keepdims=True)
    zc = z - mean
    var = jnp.mean(zc * zc, axis=0, keepdims=True)
    bn = g_ref[...] * zc * lax.rsqrt(var + EPS) + be_ref[...]
    out_ref[...] = jnp.maximum(bn + x_ref[...], 0.0)


def kernel(x, ei, batch, W1, b1, W2, b2, gamma2, beta2):
    del batch
    src = ei[0]
    dst = ei[1]
    b1r = b1.reshape(1, D)
    b2r = b2.reshape(1, D)
    gr = gamma2.reshape(1, D)
    ber = beta2.reshape(1, D)

    degp = _sc_degree()(dst)
    degp3 = degp.reshape(NC, NPAD, 1)

    y1, dinv = pl.pallas_call(
        _tc_prescale_body,
        out_shape=(
            jax.ShapeDtypeStruct((N, D), jnp.float32),
            jax.ShapeDtypeStruct((N, 1), jnp.float32),
        ),
    )(x, W1, degp3)

    p1 = _sc_segsum()(y1, src, dst)

    y2 = pl.pallas_call(
        _tc_mid_body,
        out_shape=jax.ShapeDtypeStruct((N, D), jnp.float32),
    )(p1, y1, dinv, b1r, gr, ber, W2)

    p2 = _sc_segsum()(y2, src, dst)

    out = pl.pallas_call(
        _tc_final_body,
        out_shape=jax.ShapeDtypeStruct((N, D), jnp.float32),
    )(p2, y2, x, dinv, b2r, gr, ber)
    return out


# speedup vs baseline: 13.1531x; 1.0825x over previous
import functools

import jax
import jax.numpy as jnp
from jax import lax
from jax.experimental import pallas as pl
from jax.experimental.pallas import tpu as pltpu
from jax.experimental.pallas import tpu_sc as plsc

N = 10000
E = 320000
D = 128
DH = D // 2
EPS = 1e-5

NC = 2
NS = 16
NW = NC * NS
CH = 40
EPW = E // NW
NCHUNK = EPW // CH
NCHT = E // CH
DEG_NCH = NCHT // NW
RPT = 624
RTAIL = N - NS * RPT
NPAD = 10240
DPT = NPAD // NS


def _mesh():
    return plsc.VectorSubcoreMesh(
        core_axis_name="c", subcore_axis_name="s", num_cores=NC, num_subcores=NS
    )


def _sc_degree_body(sd_hbm, degp_hbm, sd_v, ones_v, zero_v, deg_sh):
    cid = lax.axis_index("c")
    sid = lax.axis_index("s")
    wid = cid * NS + sid
    g0 = wid * DEG_NCH
    pltpu.sync_copy(sd_hbm.at[pl.ds(g0, DEG_NCH)], sd_v)
    for j in range(3):
        ones_v[pl.ds(16 * j, 16)] = jnp.ones((16,), jnp.float32)
    for j in range(DPT // 16):
        zero_v[pl.ds(16 * j, 16)] = jnp.zeros((16,), jnp.float32)
    pltpu.sync_copy(zero_v, deg_sh.at[pl.ds(sid * DPT, DPT)])
    plsc.subcore_barrier()

    def step(i, carry):
        pltpu.sync_copy(ones_v.at[pl.ds(0, CH)], deg_sh.at[sd_v.at[i, 1]], add=True)
        return carry

    lax.fori_loop(0, DEG_NCH, step, 0)
    plsc.subcore_barrier()
    pltpu.sync_copy(
        deg_sh.at[pl.ds(sid * DPT, DPT)],
        degp_hbm.at[pl.ds(cid * NPAD + sid * DPT, DPT)],
    )


@functools.cache
def _sc_degree():
    return pl.kernel(
        _sc_degree_body,
        out_type=jax.ShapeDtypeStruct((NC * NPAD,), jnp.float32),
        mesh=_mesh(),
        scratch_types=[
            pltpu.VMEM((DEG_NCH, 2, CH), jnp.int32),
            pltpu.VMEM((48,), jnp.float32),
            pltpu.VMEM((DPT,), jnp.float32),
            pltpu.VMEM_SHARED((NPAD,), jnp.float32),
        ],
    )


def _sc_segsum_body(
    y_hbm, src_hbm, dst_hbm, part_hbm,
    idxs0, idxs1, idxd0, idxd1, rows_v, acc_sh, sem_r0, sem_r1,
):
    cid = lax.axis_index("c")
    sid = lax.axis_index("s")
    wid = cid * NS + sid
    e0 = wid * EPW
    sem_r = (sem_r0, sem_r1)
    idxs = (idxs0, idxs1)
    idxd = (idxd0, idxd1)

    pltpu.sync_copy(y_hbm.at[pl.ds(sid * RPT, RPT)], acc_sh.at[pl.ds(sid * RPT, RPT)])

    @pl.when(sid == 0)
    def _():
        pltpu.sync_copy(
            y_hbm.at[pl.ds(NS * RPT, RTAIL)], acc_sh.at[pl.ds(NS * RPT, RTAIL)]
        )

    plsc.subcore_barrier()

    def fetch(i, slot):
        base = pl.multiple_of(e0 + i * CH, 8)
        pltpu.sync_copy(src_hbm.at[pl.ds(base, CH)], idxs[slot])
        pltpu.sync_copy(dst_hbm.at[pl.ds(base, CH)], idxd[slot])

    def gs(slot):
        pltpu.async_copy(y_hbm.at[idxs[slot]], rows_v.at[slot], sem_r[slot])

    def gw(slot):
        pltpu.make_async_copy(
            y_hbm.at[idxs[slot]], rows_v.at[slot], sem_r[slot]
        ).wait()

    def scat(slot):
        pltpu.sync_copy(rows_v.at[slot], acc_sh.at[idxd[slot]], add=True)

    fetch(0, 0)
    gs(0)

    def pair(p, carry):
        a = 2 * p
        fetch(a + 1, 1)
        gs(1)
        gw(0)
        scat(0)

        @pl.when(a + 2 < NCHUNK)
        def _():
            fetch(a + 2, 0)
            gs(0)

        gw(1)
        scat(1)
        return carry

    lax.fori_loop(0, NCHUNK // 2, pair, 0)
    plsc.subcore_barrier()
    pltpu.sync_copy(
        acc_sh.at[pl.ds(sid * RPT, RPT)], part_hbm.at[cid, pl.ds(sid * RPT, RPT)]
    )

    @pl.when(sid == 0)
    def _():
        pltpu.sync_copy(
            acc_sh.at[pl.ds(NS * RPT, RTAIL)], part_hbm.at[cid, pl.ds(NS * RPT, RTAIL)]
        )


@functools.cache
def _sc_segsum():
    return pl.kernel(
        _sc_segsum_body,
        out_type=jax.ShapeDtypeStruct((NC, N, D), jnp.float32),
        mesh=_mesh(),
        scratch_types=[
            pltpu.VMEM((CH,), jnp.int32),
            pltpu.VMEM((CH,), jnp.int32),
            pltpu.VMEM((CH,), jnp.int32),
            pltpu.VMEM((CH,), jnp.int32),
            pltpu.VMEM((2, CH, D), jnp.float32),
            pltpu.VMEM_SHARED((N, D), jnp.float32),
            pltpu.SemaphoreType.DMA,
            pltpu.SemaphoreType.DMA,
        ],
    )


def _tc_prescale_body(x_ref, w_ref, degp_ref, y_ref, dinv_ref):
    deg = degp_ref[0, :N] + degp_ref[1, :N] + 1.0
    dinv = lax.rsqrt(deg)
    dinv_ref[...] = dinv
    xw = jnp.dot(x_ref[...], w_ref[...], preferred_element_type=jnp.float32)
    y_ref[...] = dinv * xw


def _tc_mid_body(p_ref, y1_ref, dinv_ref, b1_ref, g_ref, be_ref, w2_ref, y_ref):
    dinv = dinv_ref[...]
    z = dinv * (p_ref[0] + p_ref[1] - y1_ref[...]) + b1_ref[...]
    mean = jnp.mean(z, axis=0, keepdims=True)
    zc = z - mean
    var = jnp.mean(zc * zc, axis=0, keepdims=True)
    h = jnp.maximum(g_ref[...] * zc * lax.rsqrt(var + EPS) + be_ref[...], 0.0)
    hw = jnp.dot(h, w2_ref[...], preferred_element_type=jnp.float32)
    y_ref[...] = dinv * hw


def _tc_final_body(p_ref, y2_ref, x_ref, dinv_ref, b2_ref, g_ref, be_ref, out_ref):
    z = dinv_ref[...] * (p_ref[0] + p_ref[1] - y2_ref[...]) + b2_ref[...]
    mean = jnp.mean(z, axis=0, keepdims=True)
    zc = z - mean
    var = jnp.mean(zc * zc, axis=0, keepdims=True)
    bn = g_ref[...] * zc * lax.rsqrt(var + EPS) + be_ref[...]
    out_ref[...] = jnp.maximum(bn + x_ref[...], 0.0)


def kernel(x, ei, batch, W1, b1, W2, b2, gamma2, beta2):
    del batch
    sd = ei.reshape(2, NCHT, CH).transpose(1, 0, 2)
    src = ei[0]
    dst = ei[1]
    b1r = b1.reshape(1, D)
    b2r = b2.reshape(1, D)
    gr = gamma2.reshape(1, D)
    ber = beta2.reshape(1, D)

    degp = _sc_degree()(sd)
    degp3 = degp.reshape(NC, NPAD, 1)

    y1, dinv = pl.pallas_call(
        _tc_prescale_body,
        out_shape=(
            jax.ShapeDtypeStruct((N, D), jnp.float32),
            jax.ShapeDtypeStruct((N, 1), jnp.float32),
        ),
    )(x, W1, degp3)

    p1 = _sc_segsum()(y1, src, dst)

    y2 = pl.pallas_call(
        _tc_mid_body,
        out_shape=jax.ShapeDtypeStruct((N, D), jnp.float32),
    )(p1, y1, dinv, b1r, gr, ber, W2)

    p2 = _sc_segsum()(y2, src, dst)

    out = pl.pallas_call(
        _tc_final_body,
        out_shape=jax.ShapeDtypeStruct((N, D), jnp.float32),
    )(p2, y2, x, dinv, b2r, gr, ber)
    return out


# speedup vs baseline: 15.7973x; 1.2010x over previous
import functools

import jax
import jax.numpy as jnp
from jax import lax
from jax.experimental import pallas as pl
from jax.experimental.pallas import tpu as pltpu
from jax.experimental.pallas import tpu_sc as plsc

N = 10000
E = 320000
D = 128
DH = D // 2
EPS = 1e-5

NC = 2
NS = 16
NW = NC * NS
CH = 40
EPW = E // NW
NCHUNK = EPW // CH
NCHT = E // CH
DEG_NCH = NCHT // NW
RPT = 624
RTAIL = N - NS * RPT
NPAD = 10240
DPT = NPAD // NS


def _mesh():
    return plsc.VectorSubcoreMesh(
        core_axis_name="c", subcore_axis_name="s", num_cores=NC, num_subcores=NS
    )


def _sc_degree_body(sd_hbm, degp_hbm, sd_v, ones_v, zero_v, deg_sh):
    cid = lax.axis_index("c")
    sid = lax.axis_index("s")
    wid = cid * NS + sid
    g0 = wid * DEG_NCH
    pltpu.sync_copy(sd_hbm.at[pl.ds(g0, DEG_NCH)], sd_v)
    for j in range(3):
        ones_v[pl.ds(16 * j, 16)] = jnp.ones((16,), jnp.float32)
    for j in range(DPT // 16):
        zero_v[pl.ds(16 * j, 16)] = jnp.zeros((16,), jnp.float32)
    pltpu.sync_copy(zero_v, deg_sh.at[pl.ds(sid * DPT, DPT)])
    plsc.subcore_barrier()

    def step(i, carry):
        pltpu.sync_copy(ones_v.at[pl.ds(0, CH)], deg_sh.at[sd_v.at[i, 1]], add=True)
        return carry

    lax.fori_loop(0, DEG_NCH, step, 0)
    plsc.subcore_barrier()
    pltpu.sync_copy(
        deg_sh.at[pl.ds(sid * DPT, DPT)],
        degp_hbm.at[pl.ds(cid * NPAD + sid * DPT, DPT)],
    )


@functools.cache
def _sc_degree():
    return pl.kernel(
        _sc_degree_body,
        out_type=jax.ShapeDtypeStruct((NC * NPAD,), jnp.float32),
        mesh=_mesh(),
        scratch_types=[
            pltpu.VMEM((DEG_NCH, 2, CH), jnp.int32),
            pltpu.VMEM((48,), jnp.float32),
            pltpu.VMEM((DPT,), jnp.float32),
            pltpu.VMEM_SHARED((NPAD,), jnp.float32),
        ],
    )


def _sc_segsum_body(
    y_hbm, src_hbm, dst_hbm, part_hbm,
    idxs0, idxs1, idxd0, idxd1, rows_v, acc_sh, sem_r0, sem_r1, sem_i0, sem_i1,
):
    cid = lax.axis_index("c")
    sid = lax.axis_index("s")
    wid = cid * NS + sid
    e0 = wid * EPW
    sem_r = (sem_r0, sem_r1)
    sem_i = (sem_i0, sem_i1)
    idxs = (idxs0, idxs1)
    idxd = (idxd0, idxd1)

    pltpu.sync_copy(y_hbm.at[pl.ds(sid * RPT, RPT)], acc_sh.at[pl.ds(sid * RPT, RPT)])

    @pl.when(sid == 0)
    def _():
        pltpu.sync_copy(
            y_hbm.at[pl.ds(NS * RPT, RTAIL)], acc_sh.at[pl.ds(NS * RPT, RTAIL)]
        )

    plsc.subcore_barrier()

    def fa(i, slot):
        base = pl.multiple_of(e0 + i * CH, 8)
        pltpu.async_copy(src_hbm.at[pl.ds(base, CH)], idxs[slot], sem_i[slot])
        pltpu.async_copy(dst_hbm.at[pl.ds(base, CH)], idxd[slot], sem_i[slot])

    def iw(i, slot):
        base = pl.multiple_of(e0 + i * CH, 8)
        pltpu.make_async_copy(
            src_hbm.at[pl.ds(base, CH)], idxs[slot], sem_i[slot]
        ).wait()
        pltpu.make_async_copy(
            dst_hbm.at[pl.ds(base, CH)], idxd[slot], sem_i[slot]
        ).wait()

    def gs(slot):
        pltpu.async_copy(y_hbm.at[idxs[slot]], rows_v.at[slot], sem_r[slot])

    def gw(slot):
        pltpu.make_async_copy(
            y_hbm.at[idxs[slot]], rows_v.at[slot], sem_r[slot]
        ).wait()

    def scat(slot):
        pltpu.sync_copy(rows_v.at[slot], acc_sh.at[idxd[slot]], add=True)

    fa(0, 0)
    fa(1, 1)
    iw(0, 0)
    gs(0)

    def pair(p, carry):
        a = 2 * p
        gw(0)
        iw(a + 1, 1)
        gs(1)
        scat(0)

        @pl.when(a + 2 < NCHUNK)
        def _():
            fa(a + 2, 0)

        gw(1)

        @pl.when(a + 2 < NCHUNK)
        def _():
            iw(a + 2, 0)
            gs(0)

        scat(1)

        @pl.when(a + 3 < NCHUNK)
        def _():
            fa(a + 3, 1)

        return carry

    lax.fori_loop(0, NCHUNK // 2, pair, 0)
    plsc.subcore_barrier()
    pltpu.sync_copy(
        acc_sh.at[pl.ds(sid * RPT, RPT)], part_hbm.at[cid, pl.ds(sid * RPT, RPT)]
    )

    @pl.when(sid == 0)
    def _():
        pltpu.sync_copy(
            acc_sh.at[pl.ds(NS * RPT, RTAIL)], part_hbm.at[cid, pl.ds(NS * RPT, RTAIL)]
        )


@functools.cache
def _sc_segsum():
    return pl.kernel(
        _sc_segsum_body,
        out_type=jax.ShapeDtypeStruct((NC, N, D), jnp.float32),
        mesh=_mesh(),
        scratch_types=[
            pltpu.VMEM((CH,), jnp.int32),
            pltpu.VMEM((CH,), jnp.int32),
            pltpu.VMEM((CH,), jnp.int32),
            pltpu.VMEM((CH,), jnp.int32),
            pltpu.VMEM((2, CH, D), jnp.float32),
            pltpu.VMEM_SHARED((N, D), jnp.float32),
            pltpu.SemaphoreType.DMA,
            pltpu.SemaphoreType.DMA,
            pltpu.SemaphoreType.DMA,
            pltpu.SemaphoreType.DMA,
        ],
    )


def _tc_prescale_body(x_ref, w_ref, degp_ref, y_ref, dinv_ref):
    deg = degp_ref[0, :N] + degp_ref[1, :N] + 1.0
    dinv = lax.rsqrt(deg)
    dinv_ref[...] = dinv
    xw = jnp.dot(x_ref[...], w_ref[...], preferred_element_type=jnp.float32)
    y_ref[...] = dinv * xw


def _tc_mid_body(p_ref, y1_ref, dinv_ref, b1_ref, g_ref, be_ref, w2_ref, y_ref):
    dinv = dinv_ref[...]
    z = dinv * (p_ref[0] + p_ref[1] - y1_ref[...]) + b1_ref[...]
    mean = jnp.mean(z, axis=0, keepdims=True)
    zc = z - mean
    var = jnp.mean(zc * zc, axis=0, keepdims=True)
    h = jnp.maximum(g_ref[...] * zc * lax.rsqrt(var + EPS) + be_ref[...], 0.0)
    hw = jnp.dot(h, w2_ref[...], preferred_element_type=jnp.float32)
    y_ref[...] = dinv * hw


def _tc_final_body(p_ref, y2_ref, x_ref, dinv_ref, b2_ref, g_ref, be_ref, out_ref):
    z = dinv_ref[...] * (p_ref[0] + p_ref[1] - y2_ref[...]) + b2_ref[...]
    mean = jnp.mean(z, axis=0, keepdims=True)
    zc = z - mean
    var = jnp.mean(zc * zc, axis=0, keepdims=True)
    bn = g_ref[...] * zc * lax.rsqrt(var + EPS) + be_ref[...]
    out_ref[...] = jnp.maximum(bn + x_ref[...], 0.0)


def kernel(x, ei, batch, W1, b1, W2, b2, gamma2, beta2):
    del batch
    sd = ei.reshape(2, NCHT, CH).transpose(1, 0, 2)
    src = ei[0]
    dst = ei[1]
    b1r = b1.reshape(1, D)
    b2r = b2.reshape(1, D)
    gr = gamma2.reshape(1, D)
    ber = beta2.reshape(1, D)

    degp = _sc_degree()(sd)
    degp3 = degp.reshape(NC, NPAD, 1)

    y1, dinv = pl.pallas_call(
        _tc_prescale_body,
        out_shape=(
            jax.ShapeDtypeStruct((N, D), jnp.float32),
            jax.ShapeDtypeStruct((N, 1), jnp.float32),
        ),
    )(x, W1, degp3)

    p1 = _sc_segsum()(y1, src, dst)

    y2 = pl.pallas_call(
        _tc_mid_body,
        out_shape=jax.ShapeDtypeStruct((N, D), jnp.float32),
    )(p1, y1, dinv, b1r, gr, ber, W2)

    p2 = _sc_segsum()(y2, src, dst)

    out = pl.pallas_call(
        _tc_final_body,
        out_shape=jax.ShapeDtypeStruct((N, D), jnp.float32),
    )(p2, y2, x, dinv, b2r, gr, ber)
    return out


# speedup vs baseline: 24.0846x; 1.5246x over previous
import functools

import jax
import jax.numpy as jnp
from jax import lax
from jax.experimental import pallas as pl
from jax.experimental.pallas import tpu as pltpu
from jax.experimental.pallas import tpu_sc as plsc

N = 10000
E = 320000
D = 128
DH = D // 2
EPS = 1e-5

NC = 2
NS = 16
NW = NC * NS
CH = 40
EPW = E // NW
SCH = 128
NFULL = EPW // SCH
TAIL = EPW - NFULL * SCH
NCHT = E // CH
DEG_NCH = NCHT // NW
RPT = 624
RTAIL = N - NS * RPT
NPAD = 10240
DPT = NPAD // NS


def _mesh():
    return plsc.VectorSubcoreMesh(
        core_axis_name="c", subcore_axis_name="s", num_cores=NC, num_subcores=NS
    )


def _sc_degree_body(sd_hbm, degp_hbm, sd_v, ones_v, zero_v, deg_sh):
    cid = lax.axis_index("c")
    sid = lax.axis_index("s")
    wid = cid * NS + sid
    g0 = wid * DEG_NCH
    pltpu.sync_copy(sd_hbm.at[pl.ds(g0, DEG_NCH)], sd_v)
    for j in range(3):
        ones_v[pl.ds(16 * j, 16)] = jnp.ones((16,), jnp.float32)
    for j in range(DPT // 16):
        zero_v[pl.ds(16 * j, 16)] = jnp.zeros((16,), jnp.float32)
    pltpu.sync_copy(zero_v, deg_sh.at[pl.ds(sid * DPT, DPT)])
    plsc.subcore_barrier()

    def step(i, carry):
        pltpu.sync_copy(ones_v.at[pl.ds(0, CH)], deg_sh.at[sd_v.at[i, 1]], add=True)
        return carry

    lax.fori_loop(0, DEG_NCH, step, 0)
    plsc.subcore_barrier()
    pltpu.sync_copy(
        deg_sh.at[pl.ds(sid * DPT, DPT)],
        degp_hbm.at[pl.ds(cid * NPAD + sid * DPT, DPT)],
    )


@functools.cache
def _sc_degree():
    return pl.kernel(
        _sc_degree_body,
        out_type=jax.ShapeDtypeStruct((NC * NPAD,), jnp.float32),
        mesh=_mesh(),
        scratch_types=[
            pltpu.VMEM((DEG_NCH, 2, CH), jnp.int32),
            pltpu.VMEM((48,), jnp.float32),
            pltpu.VMEM((DPT,), jnp.float32),
            pltpu.VMEM_SHARED((NPAD,), jnp.float32),
        ],
    )


def _sc_segsum_body(
    y_hbm, src_hbm, dst_hbm, part_hbm,
    idxs0, idxs1, idxd0, idxd1, idxt_s, idxt_d, rows_v, acc_sh,
    sem_r0, sem_r1, sem_i0, sem_i1,
):
    cid = lax.axis_index("c")
    sid = lax.axis_index("s")
    wid = cid * NS + sid
    e0 = wid * EPW
    sem_r = (sem_r0, sem_r1)
    sem_i = (sem_i0, sem_i1)
    idxs = (idxs0, idxs1)
    idxd = (idxd0, idxd1)

    pltpu.sync_copy(y_hbm.at[pl.ds(sid * RPT, RPT)], acc_sh.at[pl.ds(sid * RPT, RPT)])

    @pl.when(sid == 0)
    def _():
        pltpu.sync_copy(
            y_hbm.at[pl.ds(NS * RPT, RTAIL)], acc_sh.at[pl.ds(NS * RPT, RTAIL)]
        )

    plsc.subcore_barrier()

    def fa(i, slot):
        base = pl.multiple_of(e0 + i * SCH, 8)
        pltpu.async_copy(src_hbm.at[pl.ds(base, SCH)], idxs[slot], sem_i[slot])
        pltpu.async_copy(dst_hbm.at[pl.ds(base, SCH)], idxd[slot], sem_i[slot])

    def iw(i, slot):
        base = pl.multiple_of(e0 + i * SCH, 8)
        pltpu.make_async_copy(
            src_hbm.at[pl.ds(base, SCH)], idxs[slot], sem_i[slot]
        ).wait()
        pltpu.make_async_copy(
            dst_hbm.at[pl.ds(base, SCH)], idxd[slot], sem_i[slot]
        ).wait()

    def gs(slot):
        pltpu.async_copy(y_hbm.at[idxs[slot]], rows_v.at[slot], sem_r[slot])

    def gw(slot):
        pltpu.make_async_copy(
            y_hbm.at[idxs[slot]], rows_v.at[slot], sem_r[slot]
        ).wait()

    def scat(slot):
        pltpu.sync_copy(rows_v.at[slot], acc_sh.at[idxd[slot]], add=True)

    fa(0, 0)
    fa(1, 1)
    iw(0, 0)
    gs(0)

    def pair(p, carry):
        a = 2 * p
        gw(0)
        iw(a + 1, 1)
        gs(1)
        scat(0)

        @pl.when(a + 2 < NFULL)
        def _():
            fa(a + 2, 0)

        gw(1)

        @pl.when(a + 2 < NFULL)
        def _():
            iw(a + 2, 0)
            gs(0)

        scat(1)

        @pl.when(a + 3 < NFULL)
        def _():
            fa(a + 3, 1)

        return carry

    lax.fori_loop(0, NFULL // 2, pair, 0)

    tbase = pl.multiple_of(e0 + NFULL * SCH, 8)
    pltpu.sync_copy(src_hbm.at[pl.ds(tbase, TAIL)], idxt_s)
    pltpu.sync_copy(dst_hbm.at[pl.ds(tbase, TAIL)], idxt_d)
    pltpu.sync_copy(y_hbm.at[idxt_s], rows_v.at[0, pl.ds(0, TAIL)])
    pltpu.sync_copy(rows_v.at[0, pl.ds(0, TAIL)], acc_sh.at[idxt_d], add=True)
    plsc.subcore_barrier()
    pltpu.sync_copy(
        acc_sh.at[pl.ds(sid * RPT, RPT)], part_hbm.at[cid, pl.ds(sid * RPT, RPT)]
    )

    @pl.when(sid == 0)
    def _():
        pltpu.sync_copy(
            acc_sh.at[pl.ds(NS * RPT, RTAIL)], part_hbm.at[cid, pl.ds(NS * RPT, RTAIL)]
        )


@functools.cache
def _sc_segsum():
    return pl.kernel(
        _sc_segsum_body,
        out_type=jax.ShapeDtypeStruct((NC, N, D), jnp.float32),
        mesh=_mesh(),
        scratch_types=[
            pltpu.VMEM((SCH,), jnp.int32),
            pltpu.VMEM((SCH,), jnp.int32),
            pltpu.VMEM((SCH,), jnp.int32),
            pltpu.VMEM((SCH,), jnp.int32),
            pltpu.VMEM((TAIL,), jnp.int32),
            pltpu.VMEM((TAIL,), jnp.int32),
            pltpu.VMEM((2, SCH, D), jnp.float32),
            pltpu.VMEM_SHARED((N, D), jnp.float32),
            pltpu.SemaphoreType.DMA,
            pltpu.SemaphoreType.DMA,
            pltpu.SemaphoreType.DMA,
            pltpu.SemaphoreType.DMA,
        ],
    )


def _tc_prescale_body(x_ref, w_ref, degp_ref, y_ref, dinv_ref):
    deg = degp_ref[0, :N] + degp_ref[1, :N] + 1.0
    dinv = lax.rsqrt(deg)
    dinv_ref[...] = dinv
    xw = jnp.dot(x_ref[...], w_ref[...], preferred_element_type=jnp.float32)
    y_ref[...] = dinv * xw


def _tc_mid_body(p_ref, y1_ref, dinv_ref, b1_ref, g_ref, be_ref, w2_ref, y_ref):
    dinv = dinv_ref[...]
    z = dinv * (p_ref[0] + p_ref[1] - y1_ref[...]) + b1_ref[...]
    mean = jnp.mean(z, axis=0, keepdims=True)
    zc = z - mean
    var = jnp.mean(zc * zc, axis=0, keepdims=True)
    h = jnp.maximum(g_ref[...] * zc * lax.rsqrt(var + EPS) + be_ref[...], 0.0)
    hw = jnp.dot(h, w2_ref[...], preferred_element_type=jnp.float32)
    y_ref[...] = dinv * hw


def _tc_final_body(p_ref, y2_ref, x_ref, dinv_ref, b2_ref, g_ref, be_ref, out_ref):
    z = dinv_ref[...] * (p_ref[0] + p_ref[1] - y2_ref[...]) + b2_ref[...]
    mean = jnp.mean(z, axis=0, keepdims=True)
    zc = z - mean
    var = jnp.mean(zc * zc, axis=0, keepdims=True)
    bn = g_ref[...] * zc * lax.rsqrt(var + EPS) + be_ref[...]
    out_ref[...] = jnp.maximum(bn + x_ref[...], 0.0)


def kernel(x, ei, batch, W1, b1, W2, b2, gamma2, beta2):
    del batch
    sd = ei.reshape(2, NCHT, CH).transpose(1, 0, 2)
    src = ei[0]
    dst = ei[1]
    b1r = b1.reshape(1, D)
    b2r = b2.reshape(1, D)
    gr = gamma2.reshape(1, D)
    ber = beta2.reshape(1, D)

    degp = _sc_degree()(sd)
    degp3 = degp.reshape(NC, NPAD, 1)

    y1, dinv = pl.pallas_call(
        _tc_prescale_body,
        out_shape=(
            jax.ShapeDtypeStruct((N, D), jnp.float32),
            jax.ShapeDtypeStruct((N, 1), jnp.float32),
        ),
    )(x, W1, degp3)

    p1 = _sc_segsum()(y1, src, dst)

    y2 = pl.pallas_call(
        _tc_mid_body,
        out_shape=jax.ShapeDtypeStruct((N, D), jnp.float32),
    )(p1, y1, dinv, b1r, gr, ber, W2)

    p2 = _sc_segsum()(y2, src, dst)

    out = pl.pallas_call(
        _tc_final_body,
        out_shape=jax.ShapeDtypeStruct((N, D), jnp.float32),
    )(p2, y2, x, dinv, b2r, gr, ber)
    return out
